# baseline (speedup 1.0000x reference)
import functools

import jax
import jax.numpy as jnp
from jax import lax
from jax.experimental import pallas as pl
from jax.experimental.pallas import tpu as pltpu
from jax.experimental.pallas import tpu_sc as plsc

VOCAB = 1000000
EMSIZE = 64
B_TOTAL = 4096 * 200
LANE = 128
NW = 32
ROWS_TOTAL = B_TOTAL // LANE
ROWS_PER_W = ROWS_TOTAL // NW
CHUNK_ROWS = 4
CHUNK_IDX = CHUNK_ROWS * LANE
G = ROWS_PER_W // CHUNK_ROWS

_mesh = plsc.VectorSubcoreMesh(core_axis_name="c", subcore_axis_name="s")


@functools.partial(
    pl.kernel,
    mesh=_mesh,
    out_type=jax.ShapeDtypeStruct((B_TOTAL, EMSIZE), jnp.float32),
    scratch_types=[
        pltpu.VMEM((CHUNK_ROWS, LANE), jnp.int32),
        pltpu.VMEM((CHUNK_IDX, EMSIZE), jnp.float32),
        pltpu.SemaphoreType.DMA,
    ],
    compiler_params=pltpu.CompilerParams(use_tc_tiling_on_sc=False),
)
def _gather_kernel(idx_hbm, table_hbm, out_hbm, idx_v, rows_v, sem):
    wid = lax.axis_index("s") * 2 + lax.axis_index("c")
    row_base = wid * ROWS_PER_W

    def body(g, carry):
        r0 = row_base + g * CHUNK_ROWS
        o0 = r0 * LANE
        pltpu.sync_copy(idx_hbm.at[pl.ds(r0, CHUNK_ROWS)], idx_v)
        copies = [
            pltpu.async_copy(
                table_hbm.at[idx_v.at[j]],
                rows_v.at[pl.ds(j * LANE, LANE)],
                sem,
            )
            for j in range(CHUNK_ROWS)
        ]
        for cp in copies:
            cp.wait()
        pltpu.sync_copy(rows_v, out_hbm.at[pl.ds(o0, CHUNK_IDX)])
        return carry

    lax.fori_loop(0, G, body, 0)


def kernel(input_variable, weight):
    idx = input_variable.astype(jnp.int32).reshape(ROWS_TOTAL, LANE)
    out = _gather_kernel(idx, weight)
    return out.reshape(input_variable.shape[0], input_variable.shape[1], EMSIZE)


# speedup vs baseline: 1.0413x; 1.0413x over previous
import functools

import jax
import jax.numpy as jnp
from jax import lax
from jax.experimental import pallas as pl
from jax.experimental.pallas import tpu as pltpu
from jax.experimental.pallas import tpu_sc as plsc

VOCAB = 1000000
EMSIZE = 64
B_TOTAL = 4096 * 200
LANE = 128
NW = 32
ROWS_TOTAL = B_TOTAL // LANE
ROWS_PER_W = ROWS_TOTAL // NW
CHUNK_ROWS = 4
CHUNK_IDX = CHUNK_ROWS * LANE
G = ROWS_PER_W // CHUNK_ROWS
NBUF = 2

_mesh = plsc.VectorSubcoreMesh(core_axis_name="c", subcore_axis_name="s")


@functools.partial(
    pl.kernel,
    mesh=_mesh,
    out_type=jax.ShapeDtypeStruct((B_TOTAL, EMSIZE), jnp.float32),
    scratch_types=[
        pltpu.VMEM((NBUF, CHUNK_ROWS, LANE), jnp.int32),
        pltpu.VMEM((NBUF, CHUNK_IDX, EMSIZE), jnp.float32),
        pltpu.SemaphoreType.DMA,
        pltpu.SemaphoreType.DMA,
        pltpu.SemaphoreType.DMA,
        pltpu.SemaphoreType.DMA,
        pltpu.SemaphoreType.DMA,
        pltpu.SemaphoreType.DMA,
    ],
    compiler_params=pltpu.CompilerParams(use_tc_tiling_on_sc=False),
)
def _gather_kernel(idx_hbm, table_hbm, out_hbm, idx_v, rows_v,
                   sem_idx0, sem_idx1, sem_gat0, sem_gat1, sem_out0, sem_out1):
    sem_idx = (sem_idx0, sem_idx1)
    sem_gat = (sem_gat0, sem_gat1)
    sem_out = (sem_out0, sem_out1)
    wid = lax.axis_index("s") * 2 + lax.axis_index("c")
    row_base = wid * ROWS_PER_W

    def idx_copy(g, b):
        r0 = row_base + g * CHUNK_ROWS
        return pltpu.make_async_copy(
            idx_hbm.at[pl.ds(r0, CHUNK_ROWS)], idx_v.at[b], sem_idx[b])

    def out_copy(g, b):
        o0 = (row_base + g * CHUNK_ROWS) * LANE
        return pltpu.make_async_copy(
            rows_v.at[b], out_hbm.at[pl.ds(o0, CHUNK_IDX)], sem_out[b])

    def gather(b):
        cps = [
            pltpu.async_copy(
                table_hbm.at[idx_v.at[b, j]],
                rows_v.at[b, pl.ds(j * LANE, LANE)],
                sem_gat[b],
            )
            for j in range(CHUNK_ROWS)
        ]
        for cp in cps:
            cp.wait()

    def mid(g, b, *, first, last):
        idx_copy(g, b).wait()
        if not first:
            out_copy(g - NBUF, b).wait()
        gather(b)
        out_copy(g, b).start()
        if not last:
            idx_copy(g + NBUF, b).start()

    idx_copy(0, 0).start()
    idx_copy(1, 1).start()
    mid(0, 0, first=True, last=False)
    mid(1, 1, first=True, last=False)

    @pl.loop(1, G // NBUF - 1)
    def _steady(i):
        g0 = i * NBUF
        mid(g0, 0, first=False, last=False)
        mid(g0 + 1, 1, first=False, last=False)

    mid(G - 2, 0, first=False, last=True)
    mid(G - 1, 1, first=False, last=True)

    out_copy(G - 2, 0).wait()
    out_copy(G - 1, 1).wait()


def kernel(input_variable, weight):
    idx = input_variable.astype(jnp.int32).reshape(ROWS_TOTAL, LANE)
    out = _gather_kernel(idx, weight)
    return out.reshape(input_variable.shape[0], input_variable.shape[1], EMSIZE)


# speedup vs baseline: 1.0460x; 1.0045x over previous
import functools

import jax
import jax.numpy as jnp
from jax import lax
from jax.experimental import pallas as pl
from jax.experimental.pallas import tpu as pltpu
from jax.experimental.pallas import tpu_sc as plsc

VOCAB = 1000000
EMSIZE = 64
B_TOTAL = 4096 * 200
LANE = 128
NW = 32
ROWS_TOTAL = B_TOTAL // LANE
ROWS_PER_W = ROWS_TOTAL // NW
K = 5
BLOCK = 2 * K
N_STEADY = (ROWS_PER_W - BLOCK) // BLOCK

_mesh = plsc.VectorSubcoreMesh(core_axis_name="c", subcore_axis_name="s")


@functools.partial(
    pl.kernel,
    mesh=_mesh,
    out_type=jax.ShapeDtypeStruct((B_TOTAL, EMSIZE), jnp.float32),
    scratch_types=[
        pltpu.VMEM((ROWS_PER_W, LANE), jnp.int32),
        pltpu.VMEM((2, K, LANE, EMSIZE), jnp.float32),
        pltpu.SemaphoreType.DMA((K,)),
        pltpu.SemaphoreType.DMA((2, K)),
        pltpu.SemaphoreType.DMA,
    ],
    compiler_params=pltpu.CompilerParams(use_tc_tiling_on_sc=False),
)
def _gather_kernel(idx_hbm, table_hbm, out_hbm, idx_v, rows_v,
                   gat_sem, out_sem, idx_sem):
    wid = lax.axis_index("s") * 2 + lax.axis_index("c")
    row_base = wid * ROWS_PER_W

    pltpu.async_copy(
        idx_hbm.at[pl.ds(row_base, ROWS_PER_W)], idx_v, idx_sem).wait()

    def fire(r, s, p):
        pltpu.async_copy(
            table_hbm.at[idx_v.at[r]], rows_v.at[p, s], gat_sem.at[s])

    def gat_wait(s):
        pltpu.make_async_copy(
            table_hbm.at[idx_v.at[0]], rows_v.at[0, s], gat_sem.at[s]).wait()

    def wb(r, s, p):
        o0 = (row_base + r) * LANE
        return pltpu.make_async_copy(
            rows_v.at[p, s], out_hbm.at[pl.ds(o0, LANE)], out_sem.at[p, s])

    for s in range(K):
        fire(s, s, 0)

    for rr in range(K):
        gat_wait(rr)
        wb(rr, rr, 0).start()
        fire(rr + K, rr, 1)

    @pl.loop(0, N_STEADY)
    def _steady(i):
        r0 = K + i * BLOCK
        for rr in range(BLOCK):
            r = r0 + rr
            s = rr % K
            p = 1 - (rr // K)
            gat_wait(s)
            wb(r, s, p).start()
            wb(r, s, 1 - p).wait()
            fire(r + K, s, 1 - p)

    last = ROWS_PER_W - K
    for rr in range(K):
        r = last + rr
        gat_wait(rr)
        wb(r, rr, 1).start()

    for rr in range(K):
        wb(last - K + rr, rr, 0).wait()
        wb(last + rr, rr, 1).wait()


def kernel(input_variable, weight):
    idx = input_variable.astype(jnp.int32).reshape(ROWS_TOTAL, LANE)
    out = _gather_kernel(idx, weight)
    return out.reshape(input_variable.shape[0], input_variable.shape[1], EMSIZE)
